# baseline (speedup 1.0000x reference)
import functools

import jax
import jax.numpy as jnp
from jax import lax
from jax.experimental import pallas as pl
from jax.experimental.pallas import tpu as pltpu
from jax.experimental.pallas import tpu_sc as plsc

_NC = 2
_NS = 16
_NW = _NC * _NS
_C = 8


def kernel(inputs, P):
    B, S, D = inputs.shape
    rows_w = S // _NW
    chunks = rows_w // _C
    p2 = P[:S]

    mesh = plsc.VectorSubcoreMesh(core_axis_name="c", subcore_axis_name="s")

    @functools.partial(
        pl.kernel,
        mesh=mesh,
        out_type=jax.ShapeDtypeStruct((B, S, D), jnp.float32),
        compiler_params=pltpu.CompilerParams(use_tc_tiling_on_sc=True),
        scratch_types=[
            pltpu.VMEM((2, _C, D), jnp.float32),
            pltpu.VMEM((2, B, _C, D), jnp.float32),
            pltpu.SemaphoreType.DMA((2,)),
            pltpu.SemaphoreType.DMA((2, B)),
            pltpu.SemaphoreType.DMA((2, B)),
        ],
    )
    def sc_add(x_hbm, p_hbm, o_hbm, pbuf, xbuf, psem, xsem, osem):
        wid = lax.axis_index("s") * _NC + lax.axis_index("c")
        base = wid * rows_w

        def start_p(c, par):
            pltpu.async_copy(p_hbm.at[pl.ds(base + c * _C, _C)],
                             pbuf.at[par], psem.at[par])

        def start_x(c, par, b):
            pltpu.async_copy(x_hbm.at[b, pl.ds(base + c * _C, _C)],
                             xbuf.at[par, b], xsem.at[par, b])

        def start_out(c, par, b):
            pltpu.async_copy(xbuf.at[par, b],
                             o_hbm.at[b, pl.ds(base + c * _C, _C)],
                             osem.at[par, b])

        def wait_out(par, b):
            pltpu.make_async_copy(xbuf.at[par, b],
                                  o_hbm.at[b, pl.ds(base, _C)],
                                  osem.at[par, b]).wait()

        def wait_x(par, b):
            pltpu.make_async_copy(x_hbm.at[b, pl.ds(base, _C)],
                                  xbuf.at[par, b], xsem.at[par, b]).wait()

        def wait_p(par):
            pltpu.make_async_copy(p_hbm.at[pl.ds(base, _C)],
                                  pbuf.at[par], psem.at[par]).wait()

        start_p(0, 0)
        for b in range(B):
            start_x(0, 0, b)

        @pl.loop(0, chunks, step=2)
        def _pair(c0):
            for par in (0, 1):
                cc = c0 + par
                nxt = 1 - par

                @pl.when(cc + 1 < chunks)
                def _prefetch():
                    @pl.when(cc > 0)
                    def _drain():
                        for b in range(B):
                            wait_out(nxt, b)
                    start_p(cc + 1, nxt)
                    for b in range(B):
                        start_x(cc + 1, nxt, b)

                wait_p(par)
                for b in range(B):
                    wait_x(par, b)

                    for r in range(_C):
                        @plsc.parallel_loop(0, D, 16, unroll=8)
                        def _add(j):
                            xbuf[par, b, r, pl.ds(j, 16)] = (
                                xbuf[par, b, r, pl.ds(j, 16)]
                                + pbuf[par, r, pl.ds(j, 16)]
                            )

                    start_out(cc, par, b)

        for par in (0, 1):
            for b in range(B):
                wait_out(par, b)

    return sc_add(inputs, p2)


# speedup vs baseline: 1.2264x; 1.2264x over previous
import jax
import jax.numpy as jnp
from jax.experimental import pallas as pl

_BS = 512


def _body(x_ref, p_ref, o_ref):
    o_ref[...] = x_ref[...] + p_ref[None]


def kernel(inputs, P):
    B, S, D = inputs.shape
    p_used = P[:S]
    return pl.pallas_call(
        _body,
        grid=(S // _BS, B),
        in_specs=[
            pl.BlockSpec((1, _BS, D), lambda i, j: (j, i, 0)),
            pl.BlockSpec((_BS, D), lambda i, j: (i, 0)),
        ],
        out_specs=pl.BlockSpec((1, _BS, D), lambda i, j: (j, i, 0)),
        out_shape=jax.ShapeDtypeStruct((B, S, D), inputs.dtype),
    )(inputs, p_used)
